# initial kernel scaffold (unmeasured)
import jax
import jax.numpy as jnp
from jax import lax
from jax.experimental import pallas as pl
from jax.experimental.pallas import tpu as pltpu

N_DEV = 4

_GELU_C = 0.7978845608028654


def _gelu(y):
    return 0.5 * y * (1.0 + jnp.tanh(_GELU_C * (y + 0.044715 * y * y * y)))


def kernel(x, w_mat):
    m_per, k = x.shape
    _, n = w_mat.shape
    n_per = n // N_DEV

    def body(x_ref, w_ref, out_ref, send_buf, recv_buf, send_sems, recv_sems):
        me = lax.axis_index("i")

        barrier_sem = pltpu.get_barrier_semaphore()
        for d in range(1, N_DEV):
            pl.semaphore_signal(
                barrier_sem,
                inc=1,
                device_id=((me + d) % N_DEV,),
                device_id_type=pl.DeviceIdType.MESH,
            )
        pl.semaphore_wait(barrier_sem, N_DEV - 1)

        for c in range(N_DEV):
            y = _gelu(
                jnp.dot(
                    x_ref[...],
                    w_ref[:, c * n_per : (c + 1) * n_per],
                    preferred_element_type=jnp.float32,
                )
            )

            @pl.when(me == c)
            def _():
                out_ref[pl.ds(c * m_per, m_per), :] = y

            @pl.when(me != c)
            def _():
                send_buf[c] = y.astype(jnp.bfloat16)
                rdma = pltpu.make_async_remote_copy(
                    src_ref=send_buf.at[c],
                    dst_ref=recv_buf.at[me],
                    send_sem=send_sems.at[c],
                    recv_sem=recv_sems.at[me],
                    device_id=(c,),
                    device_id_type=pl.DeviceIdType.MESH,
                )
                rdma.start()

        for s in range(N_DEV):

            @pl.when(me != s)
            def _():
                recv = pltpu.make_async_remote_copy(
                    src_ref=send_buf.at[s],
                    dst_ref=recv_buf.at[s],
                    send_sem=send_sems.at[s],
                    recv_sem=recv_sems.at[s],
                    device_id=(s,),
                    device_id_type=pl.DeviceIdType.MESH,
                )
                recv.wait_recv()
                out_ref[pl.ds(s * m_per, m_per), :] = recv_buf[s].astype(
                    jnp.float32
                )

        for c in range(N_DEV):

            @pl.when(me != c)
            def _():
                drain = pltpu.make_async_remote_copy(
                    src_ref=send_buf.at[c],
                    dst_ref=recv_buf.at[me],
                    send_sem=send_sems.at[c],
                    recv_sem=recv_sems.at[me],
                    device_id=(c,),
                    device_id_type=pl.DeviceIdType.MESH,
                )
                drain.wait_send()

    return pl.pallas_call(
        body,
        out_shape=jax.ShapeDtypeStruct((N_DEV * m_per, n_per), jnp.float32),
        in_specs=[
            pl.BlockSpec(memory_space=pltpu.VMEM),
            pl.BlockSpec(memory_space=pltpu.VMEM),
        ],
        out_specs=pl.BlockSpec(memory_space=pltpu.VMEM),
        scratch_shapes=[
            pltpu.VMEM((N_DEV, m_per, n_per), jnp.bfloat16),
            pltpu.VMEM((N_DEV, m_per, n_per), jnp.bfloat16),
            pltpu.SemaphoreType.DMA((N_DEV,)),
            pltpu.SemaphoreType.DMA((N_DEV,)),
        ],
        compiler_params=pltpu.CompilerParams(
            collective_id=0,
            vmem_limit_bytes=100 * 1024 * 1024,
        ),
    )(x, w_mat)


# baseline (device time: 64423 ns/iter reference)
import jax
import jax.numpy as jnp
from jax import lax
from jax.experimental import pallas as pl
from jax.experimental.pallas import tpu as pltpu

N_DEV = 4
XCH = 4

_GELU_C = 0.7978845608028654


def _gelu(y):
    return 0.5 * y * (1.0 + jnp.tanh(_GELU_C * (y + 0.044715 * y * y * y)))


def kernel(x, w_mat):
    m_per, k = x.shape
    _, n = w_mat.shape
    n_per = n // N_DEV
    xrows = m_per // XCH

    def body(
        x_hbm,
        w_hbm,
        out_ref,
        x_bf,
        w_bf,
        xs,
        ws,
        x_sems,
        w_sems,
        send_buf,
        recv_buf,
        send_sems,
        recv_sems,
    ):
        me = lax.axis_index("i")

        def xdma(ci, slot):
            return pltpu.make_async_copy(
                x_hbm.at[pl.ds(ci * xrows, xrows), :],
                xs.at[slot],
                x_sems.at[slot],
            )

        def wdma(c, slot):
            return pltpu.make_async_copy(
                w_hbm.at[:, pl.ds(c * n_per, n_per)],
                ws.at[slot],
                w_sems.at[slot],
            )

        wdma(0, 0).start()
        xdma(0, 0).start()

        barrier_sem = pltpu.get_barrier_semaphore()
        for d in range(1, N_DEV):
            pl.semaphore_signal(
                barrier_sem,
                inc=1,
                device_id=((me + d) % N_DEV,),
                device_id_type=pl.DeviceIdType.MESH,
            )
        pl.semaphore_wait(barrier_sem, N_DEV - 1)

        for ci in range(XCH):
            slot = ci % 2
            if ci + 1 < XCH:
                xdma(ci + 1, 1 - slot).start()
            xdma(ci, slot).wait()
            x_bf[pl.ds(ci * xrows, xrows), :] = xs[slot].astype(jnp.bfloat16)

        for c in range(N_DEV):
            slot = c % 2
            if c + 1 < N_DEV:
                wdma(c + 1, 1 - slot).start()
            wdma(c, slot).wait()
            w_bf[slot] = ws[slot].astype(jnp.bfloat16)

            y = _gelu(
                jnp.dot(
                    x_bf[...],
                    w_bf[slot],
                    preferred_element_type=jnp.float32,
                )
            )

            @pl.when(me == c)
            def _():
                out_ref[pl.ds(c * m_per, m_per), :] = y

            @pl.when(me != c)
            def _():
                send_buf[c] = y.astype(jnp.bfloat16)
                rdma = pltpu.make_async_remote_copy(
                    src_ref=send_buf.at[c],
                    dst_ref=recv_buf.at[me],
                    send_sem=send_sems.at[c],
                    recv_sem=recv_sems.at[me],
                    device_id=(c,),
                    device_id_type=pl.DeviceIdType.MESH,
                )
                rdma.start()

        for s in range(N_DEV):

            @pl.when(me != s)
            def _():
                recv = pltpu.make_async_remote_copy(
                    src_ref=send_buf.at[s],
                    dst_ref=recv_buf.at[s],
                    send_sem=send_sems.at[s],
                    recv_sem=recv_sems.at[s],
                    device_id=(s,),
                    device_id_type=pl.DeviceIdType.MESH,
                )
                recv.wait_recv()
                out_ref[pl.ds(s * m_per, m_per), :] = recv_buf[s].astype(
                    jnp.float32
                )

        for c in range(N_DEV):

            @pl.when(me != c)
            def _():
                drain = pltpu.make_async_remote_copy(
                    src_ref=send_buf.at[c],
                    dst_ref=recv_buf.at[me],
                    send_sem=send_sems.at[c],
                    recv_sem=recv_sems.at[me],
                    device_id=(c,),
                    device_id_type=pl.DeviceIdType.MESH,
                )
                drain.wait_send()

    return pl.pallas_call(
        body,
        out_shape=jax.ShapeDtypeStruct((N_DEV * m_per, n_per), jnp.float32),
        in_specs=[
            pl.BlockSpec(memory_space=pltpu.MemorySpace.HBM),
            pl.BlockSpec(memory_space=pltpu.MemorySpace.HBM),
        ],
        out_specs=pl.BlockSpec(memory_space=pltpu.VMEM),
        scratch_shapes=[
            pltpu.VMEM((m_per, k), jnp.bfloat16),
            pltpu.VMEM((2, k, n_per), jnp.bfloat16),
            pltpu.VMEM((2, xrows, k), jnp.float32),
            pltpu.VMEM((2, k, n_per), jnp.float32),
            pltpu.SemaphoreType.DMA((2,)),
            pltpu.SemaphoreType.DMA((2,)),
            pltpu.VMEM((N_DEV, m_per, n_per), jnp.bfloat16),
            pltpu.VMEM((N_DEV, m_per, n_per), jnp.bfloat16),
            pltpu.SemaphoreType.DMA((N_DEV,)),
            pltpu.SemaphoreType.DMA((N_DEV,)),
        ],
        compiler_params=pltpu.CompilerParams(
            collective_id=0,
            vmem_limit_bytes=100 * 1024 * 1024,
        ),
    )(x, w_mat)


# device time: 56746 ns/iter; 1.1353x vs baseline; 1.1353x over previous
import jax
import jax.numpy as jnp
from jax import lax
from jax.experimental import pallas as pl
from jax.experimental.pallas import tpu as pltpu

N_DEV = 4
XCH = 4

_GELU_C = 0.7978845608028654


def _gelu(y):
    return 0.5 * y * (1.0 + jnp.tanh(_GELU_C * (y + 0.044715 * y * y * y)))


def kernel(x, w_mat):
    m_per, k = x.shape
    _, n = w_mat.shape
    n_per = n // N_DEV
    xrows = m_per // XCH

    def body(
        x_hbm,
        w_hbm,
        out_ref,
        x_bf,
        w_bf,
        xs,
        ws,
        x_sems,
        w_sems,
        send_buf,
        recv_buf,
        send_sems,
        recv_sems,
    ):
        me = lax.axis_index("i")

        def xdma(ci, slot):
            return pltpu.make_async_copy(
                x_hbm.at[pl.ds(ci * xrows, xrows), :],
                xs.at[slot],
                x_sems.at[slot],
            )

        pltpu.make_async_copy(
            w_hbm.at[:, pl.ds(((me + 1) % N_DEV) * n_per, n_per)],
            ws.at[0],
            w_sems.at[0],
        ).start()
        xdma(0, 0).start()

        barrier_sem = pltpu.get_barrier_semaphore()
        for d in range(1, N_DEV):
            pl.semaphore_signal(
                barrier_sem,
                inc=1,
                device_id=((me + d) % N_DEV,),
                device_id_type=pl.DeviceIdType.MESH,
            )
        pl.semaphore_wait(barrier_sem, N_DEV - 1)

        for ci in range(XCH):
            slot = ci % 2
            if ci + 1 < XCH:
                xdma(ci + 1, 1 - slot).start()
            xdma(ci, slot).wait()
            x_bf[pl.ds(ci * xrows, xrows), :] = xs[slot].astype(jnp.bfloat16)

        offs = [1, 3, 2, 0]
        for idx, off in enumerate(offs):
            c = (me + off) % N_DEV
            slot = idx % 2
            if idx + 1 < N_DEV:
                nxt = (me + offs[idx + 1]) % N_DEV
                pltpu.make_async_copy(
                    w_hbm.at[:, pl.ds(nxt * n_per, n_per)],
                    ws.at[1 - slot],
                    w_sems.at[1 - slot],
                ).start()
            pltpu.make_async_copy(
                w_hbm.at[:, pl.ds(c * n_per, n_per)],
                ws.at[slot],
                w_sems.at[slot],
            ).wait()
            w_bf[slot] = ws[slot].astype(jnp.bfloat16)

            y = _gelu(
                jnp.dot(
                    x_bf[...],
                    w_bf[slot],
                    preferred_element_type=jnp.float32,
                )
            )

            if off == 0:
                out_ref[pl.ds(me * m_per, m_per), :] = y
            else:
                send_buf[idx] = y.astype(jnp.bfloat16)
                rdma = pltpu.make_async_remote_copy(
                    src_ref=send_buf.at[idx],
                    dst_ref=recv_buf.at[me],
                    send_sem=send_sems.at[idx],
                    recv_sem=recv_sems.at[me],
                    device_id=(c,),
                    device_id_type=pl.DeviceIdType.MESH,
                )
                rdma.start()

        for off in [3, 2, 1]:
            s = (me + off) % N_DEV
            recv = pltpu.make_async_remote_copy(
                src_ref=send_buf.at[0],
                dst_ref=recv_buf.at[s],
                send_sem=send_sems.at[0],
                recv_sem=recv_sems.at[s],
                device_id=(s,),
                device_id_type=pl.DeviceIdType.MESH,
            )
            recv.wait_recv()
            out_ref[pl.ds(s * m_per, m_per), :] = recv_buf[s].astype(
                jnp.float32
            )

        for idx, off in enumerate(offs[:3]):
            c = (me + off) % N_DEV
            drain = pltpu.make_async_remote_copy(
                src_ref=send_buf.at[idx],
                dst_ref=recv_buf.at[me],
                send_sem=send_sems.at[idx],
                recv_sem=recv_sems.at[me],
                device_id=(c,),
                device_id_type=pl.DeviceIdType.MESH,
            )
            drain.wait_send()

    return pl.pallas_call(
        body,
        out_shape=jax.ShapeDtypeStruct((N_DEV * m_per, n_per), jnp.float32),
        in_specs=[
            pl.BlockSpec(memory_space=pltpu.MemorySpace.HBM),
            pl.BlockSpec(memory_space=pltpu.MemorySpace.HBM),
        ],
        out_specs=pl.BlockSpec(memory_space=pltpu.VMEM),
        scratch_shapes=[
            pltpu.VMEM((m_per, k), jnp.bfloat16),
            pltpu.VMEM((2, k, n_per), jnp.bfloat16),
            pltpu.VMEM((2, xrows, k), jnp.float32),
            pltpu.VMEM((2, k, n_per), jnp.float32),
            pltpu.SemaphoreType.DMA((2,)),
            pltpu.SemaphoreType.DMA((2,)),
            pltpu.VMEM((N_DEV, m_per, n_per), jnp.bfloat16),
            pltpu.VMEM((N_DEV, m_per, n_per), jnp.bfloat16),
            pltpu.SemaphoreType.DMA((N_DEV,)),
            pltpu.SemaphoreType.DMA((N_DEV,)),
        ],
        compiler_params=pltpu.CompilerParams(
            collective_id=0,
            vmem_limit_bytes=100 * 1024 * 1024,
        ),
    )(x, w_mat)


# device time: 27118 ns/iter; 2.3757x vs baseline; 2.0926x over previous
import os

import jax
import jax.numpy as jnp
from jax import lax
from jax.experimental import pallas as pl
from jax.experimental.pallas import tpu as pltpu

ABLATE = int(os.environ.get("ABLATE", "0"))

N_DEV = 4
XCH = 4

_GELU_C = 0.7978845608028654


def _gelu(y):
    return 0.5 * y * (1.0 + jnp.tanh(_GELU_C * (y + 0.044715 * y * y * y)))


def kernel(x, w_mat):
    m_per, k = x.shape
    _, n = w_mat.shape
    n_per = n // N_DEV
    xrows = m_per // XCH

    def body(
        x_hbm,
        w_hbm,
        out_ref,
        x_bf,
        w_bf,
        xs,
        ws,
        x_sems,
        w_sems,
        send_buf,
        recv_buf,
        send_sems,
        recv_sems,
    ):
        me = lax.axis_index("i")

        def xdma(ci, slot):
            return pltpu.make_async_copy(
                x_hbm.at[pl.ds(ci * xrows, xrows), :],
                xs.at[slot],
                x_sems.at[slot],
            )

        pltpu.make_async_copy(
            w_hbm.at[:, pl.ds(((me + 1) % N_DEV) * n_per, n_per)],
            ws.at[0],
            w_sems.at[0],
        ).start()
        xdma(0, 0).start()

        barrier_sem = pltpu.get_barrier_semaphore()
        for d in range(1, N_DEV):
            pl.semaphore_signal(
                barrier_sem,
                inc=1,
                device_id=((me + d) % N_DEV,),
                device_id_type=pl.DeviceIdType.MESH,
            )
        pl.semaphore_wait(barrier_sem, N_DEV - 1)

        for ci in range(XCH):
            slot = ci % 2
            if ci + 1 < XCH:
                xdma(ci + 1, 1 - slot).start()
            xdma(ci, slot).wait()
            x_bf[pl.ds(ci * xrows, xrows), :] = xs[slot].astype(jnp.bfloat16)

        offs = [1, 3, 2, 0]
        for idx, off in enumerate(offs):
            c = (me + off) % N_DEV
            slot = idx % 2
            if idx + 1 < N_DEV:
                nxt = (me + offs[idx + 1]) % N_DEV
                pltpu.make_async_copy(
                    w_hbm.at[:, pl.ds(nxt * n_per, n_per)],
                    ws.at[1 - slot],
                    w_sems.at[1 - slot],
                ).start()
            pltpu.make_async_copy(
                w_hbm.at[:, pl.ds(c * n_per, n_per)],
                ws.at[slot],
                w_sems.at[slot],
            ).wait()
            w_bf[slot] = ws[slot].astype(jnp.bfloat16)

            if ABLATE >= 3:
                y = w_bf[slot][:m_per].astype(jnp.float32)
            else:
                y = jnp.dot(
                    x_bf[...],
                    w_bf[slot],
                    preferred_element_type=jnp.float32,
                )
                if ABLATE < 2:
                    y = _gelu(y)

            if off == 0:
                out_ref[pl.ds(me * m_per, m_per), :] = y
            else:
                send_buf[idx] = y.astype(jnp.bfloat16)
                if ABLATE == 0:
                    rdma = pltpu.make_async_remote_copy(
                        src_ref=send_buf.at[idx],
                        dst_ref=recv_buf.at[me],
                        send_sem=send_sems.at[idx],
                        recv_sem=recv_sems.at[me],
                        device_id=(c,),
                        device_id_type=pl.DeviceIdType.MESH,
                    )
                    rdma.start()

        for off in [3, 2, 1]:
            s = (me + off) % N_DEV
            if ABLATE == 0:
                recv = pltpu.make_async_remote_copy(
                    src_ref=send_buf.at[0],
                    dst_ref=recv_buf.at[s],
                    send_sem=send_sems.at[0],
                    recv_sem=recv_sems.at[s],
                    device_id=(s,),
                    device_id_type=pl.DeviceIdType.MESH,
                )
                recv.wait_recv()
            out_ref[pl.ds(s * m_per, m_per), :] = recv_buf[s].astype(
                jnp.float32
            )

        if ABLATE == 0:
            for idx, off in enumerate(offs[:3]):
                c = (me + off) % N_DEV
                drain = pltpu.make_async_remote_copy(
                    src_ref=send_buf.at[idx],
                    dst_ref=recv_buf.at[me],
                    send_sem=send_sems.at[idx],
                    recv_sem=recv_sems.at[me],
                    device_id=(c,),
                    device_id_type=pl.DeviceIdType.MESH,
                )
                drain.wait_send()

    return pl.pallas_call(
        body,
        out_shape=jax.ShapeDtypeStruct((N_DEV * m_per, n_per), jnp.float32),
        in_specs=[
            pl.BlockSpec(memory_space=pltpu.MemorySpace.HBM),
            pl.BlockSpec(memory_space=pltpu.MemorySpace.HBM),
        ],
        out_specs=pl.BlockSpec(memory_space=pltpu.VMEM),
        scratch_shapes=[
            pltpu.VMEM((m_per, k), jnp.bfloat16),
            pltpu.VMEM((2, k, n_per), jnp.bfloat16),
            pltpu.VMEM((2, xrows, k), jnp.float32),
            pltpu.VMEM((2, k, n_per), jnp.float32),
            pltpu.SemaphoreType.DMA((2,)),
            pltpu.SemaphoreType.DMA((2,)),
            pltpu.VMEM((N_DEV, m_per, n_per), jnp.bfloat16),
            pltpu.VMEM((N_DEV, m_per, n_per), jnp.bfloat16),
            pltpu.SemaphoreType.DMA((N_DEV,)),
            pltpu.SemaphoreType.DMA((N_DEV,)),
        ],
        compiler_params=pltpu.CompilerParams(
            collective_id=0,
            vmem_limit_bytes=100 * 1024 * 1024,
        ),
    )(x, w_mat)
